# baseline (speedup 1.0000x reference)
import jax
import jax.numpy as jnp
from jax import lax
from jax.experimental import pallas as pl
from jax.experimental.pallas import tpu as pltpu
from jax.experimental.pallas import tpu_sc as plsc

NUM_ROWS = 16384
NUM_COLS = 1000

_info = plsc.get_sparse_core_info()
NC = _info.num_cores
NS = _info.num_subcores
L = _info.num_lanes
NW = NC * NS

ROWS_PER_W = NUM_ROWS // NW
CHUNK_ROWS = 32
NCHUNK = ROWS_PER_W // CHUNK_ROWS
GROUPS = CHUNK_ROWS // L
_FILL_STARTS = tuple(range(0, NUM_COLS - L, L)) + (NUM_COLS - L,)


def _body(x_hbm, out_hbm, buf0, buf1, idx_v, sem0, sem1):
    cid = lax.axis_index("c")
    sid = lax.axis_index("s")
    wid = sid * NC + cid
    base_row = wid * ROWS_PER_W

    zeros16 = jnp.zeros((L,), jnp.float32)
    ones16 = jnp.ones((L,), jnp.float32)
    iota16 = lax.iota(jnp.int32, L)

    def _zero(r, carry):
        for c0 in _FILL_STARTS:
            buf0[r, pl.ds(c0, L)] = zeros16
            buf1[r, pl.ds(c0, L)] = zeros16
        return carry

    lax.fori_loop(0, CHUNK_ROWS, _zero, 0)

    bufs = (buf0, buf1)
    sems = (sem0, sem1)
    handles = [None, None]
    old_cols = [None, None]

    for k in range(NCHUNK):
        slot = k % 2
        buf = bufs[slot]
        row0 = base_row + k * CHUNK_ROWS

        if handles[slot] is not None:
            handles[slot].wait()
            for g in range(GROUPS):
                colv = old_cols[slot][g]
                plsc.store_scatter(buf, [iota16 + g * L, colv], zeros16,
                                   mask=colv < NUM_COLS)

        pltpu.sync_copy(x_hbm.at[pl.ds(row0, CHUNK_ROWS)], idx_v)
        cols = []
        for g in range(GROUPS):
            colv = idx_v[pl.ds(g * L, L)]
            plsc.store_scatter(buf, [iota16 + g * L, colv], ones16,
                               mask=colv < NUM_COLS)
            cols.append(colv)
        old_cols[slot] = cols

        handles[slot] = pltpu.async_copy(
            buf, out_hbm.at[pl.ds(row0, CHUNK_ROWS)], sems[slot])

    handles[0].wait()
    handles[1].wait()


@jax.jit
def kernel(x):
    mesh = plsc.VectorSubcoreMesh(core_axis_name="c", subcore_axis_name="s")
    return pl.kernel(
        _body,
        out_type=jax.ShapeDtypeStruct((NUM_ROWS, NUM_COLS), jnp.float32),
        mesh=mesh,
        compiler_params=pltpu.CompilerParams(needs_layout_passes=False,
                                             use_tc_tiling_on_sc=True),
        scratch_types=[
            pltpu.VMEM((CHUNK_ROWS, NUM_COLS), jnp.float32),
            pltpu.VMEM((CHUNK_ROWS, NUM_COLS), jnp.float32),
            pltpu.VMEM((CHUNK_ROWS,), jnp.int32),
            pltpu.SemaphoreType.DMA,
            pltpu.SemaphoreType.DMA,
        ],
    )(x.astype(jnp.int32))


# speedup vs baseline: 2.0858x; 2.0858x over previous
import jax
import jax.numpy as jnp
from jax import lax
from jax.experimental import pallas as pl
from jax.experimental.pallas import tpu as pltpu
from jax.experimental.pallas import tpu_sc as plsc

NUM_SAMPLES = 16384
NUM_CLASSES = 1000

_info = plsc.get_sparse_core_info()
NC = _info.num_cores
NS = _info.num_subcores
L = _info.num_lanes
NW = NC * NS

NCR = 16
COLS = NUM_SAMPLES // NCR
CB = 40
NCB = NUM_CLASSES // CB
MAX_J = 13
GROUPS = COLS // L


def _body(x_hbm, out_hbm, buf0, buf1, xi_v, cid_v, srw_v, sem0, sem1):
    c_idx = lax.axis_index("c")
    s_idx = lax.axis_index("s")
    w = s_idx * NC + c_idx
    cr = w % NCR
    cb0 = w // NCR
    c0 = cr * COLS

    zeros16 = jnp.zeros((L,), jnp.float32)
    ones16 = jnp.ones((L,), jnp.float32)
    iota16 = lax.iota(jnp.int32, L)

    pltpu.sync_copy(x_hbm.at[pl.ds(c0, COLS)], xi_v)

    def _pre(g, carry):
        xv = xi_v[pl.ds(g * L, L)]
        cid = xv // CB
        cid_v[pl.ds(g * L, L)] = cid
        srw_v[pl.ds(g * L, L)] = xv - cid * CB
        return carry

    lax.fori_loop(0, GROUPS, _pre, 0, unroll=4)

    def _zero(g, carry):
        for s in range(CB):
            buf0[s, pl.ds(g * L, L)] = zeros16
            buf1[s, pl.ds(g * L, L)] = zeros16
        return carry

    lax.fori_loop(0, GROUPS, _zero, 0)

    bufs = (buf0, buf1)
    sems = (sem0, sem1)

    def _scan(buf, cb):
        def body(g, carry):
            cid = cid_v[pl.ds(g * L, L)]
            srw = srw_v[pl.ds(g * L, L)]
            scol = iota16 + g * L
            m_new = cid == cb
            m_old = cid == cb - 4
            val = jnp.where(m_new, ones16, zeros16)
            plsc.store_scatter(buf, [srw, scol], val, mask=m_new | m_old)
            return carry

        lax.fori_loop(0, GROUPS, body, 0, unroll=4)

    def _chunk(j):
        slot = j % 2
        cb = cb0 + 2 * j
        _scan(bufs[slot], cb)
        return pltpu.async_copy(
            bufs[slot],
            out_hbm.at[pl.ds(cb * CB, CB), pl.ds(c0, COLS)],
            sems[slot])

    desc = [None, None]
    for j in range(MAX_J - 1):
        if desc[j % 2] is not None:
            desc[j % 2].wait()
        desc[j % 2] = _chunk(j)

    @pl.when(cb0 + 2 * (MAX_J - 1) < NCB)
    def _():
        desc[(MAX_J - 1) % 2].wait()
        _chunk(MAX_J - 1)

    desc[1].wait()
    pltpu.make_async_copy(
        buf0, out_hbm.at[pl.ds(0, CB), pl.ds(c0, COLS)], sem0).wait()


@jax.jit
def kernel(x):
    mesh = plsc.VectorSubcoreMesh(core_axis_name="c", subcore_axis_name="s")
    out_t = pl.kernel(
        _body,
        out_type=jax.ShapeDtypeStruct((NUM_CLASSES, NUM_SAMPLES), jnp.float32),
        mesh=mesh,
        compiler_params=pltpu.CompilerParams(needs_layout_passes=False,
                                             use_tc_tiling_on_sc=True),
        scratch_types=[
            pltpu.VMEM((CB, COLS), jnp.float32),
            pltpu.VMEM((CB, COLS), jnp.float32),
            pltpu.VMEM((COLS,), jnp.int32),
            pltpu.VMEM((COLS,), jnp.int32),
            pltpu.VMEM((COLS,), jnp.int32),
            pltpu.SemaphoreType.DMA,
            pltpu.SemaphoreType.DMA,
        ],
    )(x.astype(jnp.int32))
    return out_t.T
